# baseline (speedup 1.0000x reference)
import jax
import jax.numpy as jnp
from jax import lax
from jax.experimental import pallas as pl
from jax.experimental.pallas import tpu as pltpu
from jax.experimental.pallas import tpu_sc as plsc

N_NODES = 10000
N_EDGES = 320000
D_IN = 128
D_HID = 128
D_OUT = 112
D = 128

NC = 2
NS = 16
NW = NC * NS
NA = 10240
TRASH = 10200
RCH = 64
RCHUNKS = 10
CHUNK = 96
NCHUNKS = 106
EPW = CHUNK * NCHUNKS
E_PAD = EPW * NW


def _fill(ref, nrows, ncols, val):
    v = jnp.full((16,), val, jnp.float32)

    def body(r, _):
        for j in range(ncols // 16):
            ref[r, pl.ds(j * 16, 16)] = v
        return 0

    lax.fori_loop(0, nrows, body, 0)


def _make_agg(with_counts):
    out_type = [jax.ShapeDtypeStruct((NC, NA, D), jnp.float32)]
    if with_counts:
        out_type.append(jax.ShapeDtypeStruct((NC, NA, D), jnp.float32))

    scratch = [
        pltpu.VMEM((CHUNK,), jnp.int32),
        pltpu.VMEM((CHUNK,), jnp.int32),
        pltpu.VMEM((CHUNK,), jnp.int32),
        pltpu.VMEM((CHUNK,), jnp.int32),
        pltpu.VMEM((CHUNK, D), jnp.float32),
        pltpu.VMEM((CHUNK, D), jnp.float32),
        pltpu.VMEM_SHARED((NA, D), jnp.float32),
        pltpu.SemaphoreType.DMA,
        pltpu.SemaphoreType.DMA,
        pltpu.SemaphoreType.DMA,
        pltpu.SemaphoreType.DMA,
    ]

    def body(table, src, dst, *rest):
        if with_counts:
            (sum_out, cnt_out, src_v0, src_v1, dst_v0, dst_v1, rows0, rows1,
             acc, gsem0, gsem1, ssem0, ssem1) = rest
        else:
            (sum_out, src_v0, src_v1, dst_v0, dst_v1, rows0, rows1,
             acc, gsem0, gsem1, ssem0, ssem1) = rest
        src_v = (src_v0, src_v1)
        dst_v = (dst_v0, dst_v1)
        rows = (rows0, rows1)
        gsem = (gsem0, gsem1)
        ssem = (ssem0, ssem1)

        sid = lax.axis_index("s")
        cid = lax.axis_index("c")
        wid = cid * NS + sid
        ebase = wid * EPW

        def zero_acc():
            for k in range(RCHUNKS):
                roff = (sid * RCHUNKS + k) * RCH
                pltpu.sync_copy(rows0.at[pl.ds(0, RCH)],
                                acc.at[pl.ds(roff, RCH)])

        def write_acc(out):
            for k in range(RCHUNKS):
                roff = (sid * RCHUNKS + k) * RCH
                pltpu.sync_copy(acc.at[pl.ds(roff, RCH)],
                                rows0.at[pl.ds(0, RCH)])
                pltpu.sync_copy(rows0.at[pl.ds(0, RCH)],
                                out.at[cid, pl.ds(roff, RCH)])

        def wait_scatter(b):
            pltpu.make_async_copy(rows[b], acc.at[dst_v[b]], ssem[b]).wait()

        def wait_gather(b):
            pltpu.make_async_copy(table.at[src_v[b]], rows[b], gsem[b]).wait()

        _fill(rows0, CHUNK, D, 0.0)
        zero_acc()
        plsc.subcore_barrier()

        if with_counts:
            _fill(rows0, CHUNK, D, 1.0)

            def cnt_wait(b):
                pltpu.make_async_copy(rows0, acc.at[dst_v[b]], ssem[b]).wait()

            def cnt_issue(c, b):
                off = ebase + c * CHUNK
                pltpu.sync_copy(dst.at[pl.ds(off, CHUNK)], dst_v[b])
                pltpu.async_copy(rows0, acc.at[dst_v[b]], ssem[b], add=True)

            cnt_issue(0, 0)
            cnt_issue(1, 1)

            def cnt_body(c2, _):
                for b in range(2):
                    i = 2 * c2 + 2 + b
                    cnt_wait(b)
                    cnt_issue(i, b)
                return 0

            lax.fori_loop(0, NCHUNKS // 2 - 1, cnt_body, 0)
            cnt_wait(0)
            cnt_wait(1)
            plsc.subcore_barrier()
            write_acc(cnt_out)
            _fill(rows0, CHUNK, D, 0.0)
            zero_acc()
            plsc.subcore_barrier()

        def start_gather(c, b):
            off = ebase + c * CHUNK
            pltpu.sync_copy(src.at[pl.ds(off, CHUNK)], src_v[b])
            pltpu.sync_copy(dst.at[pl.ds(off, CHUNK)], dst_v[b])
            pltpu.async_copy(table.at[src_v[b]], rows[b], gsem[b])

        def start_scatter(b):
            pltpu.async_copy(rows[b], acc.at[dst_v[b]], ssem[b], add=True)

        start_gather(0, 0)
        wait_gather(0)
        start_scatter(0)
        start_gather(1, 1)

        def feat_body(c2, _):
            for b in range(2):
                i = 2 * c2 + 2 + b
                nb = 1 - b
                wait_gather(nb)
                start_scatter(nb)
                wait_scatter(b)
                start_gather(i, b)
            return 0

        lax.fori_loop(0, NCHUNKS // 2 - 1, feat_body, 0)
        wait_gather(1)
        start_scatter(1)
        wait_scatter(0)
        wait_scatter(1)
        plsc.subcore_barrier()
        write_acc(sum_out)

    mesh = plsc.VectorSubcoreMesh(core_axis_name="c", subcore_axis_name="s")
    return pl.kernel(body, out_type=out_type, mesh=mesh,
                     scratch_types=scratch)


_agg_with_counts = _make_agg(True)
_agg_no_counts = _make_agg(False)


BN = 256


def _layer1_tc(sum_ref, inv_ref, x_ref, wl_ref, bl_ref, wr_ref, out_ref):
    mean = (sum_ref[0] + sum_ref[1]) * inv_ref[...]
    h = (jnp.dot(mean, wl_ref[...], preferred_element_type=jnp.float32)
         + jnp.dot(x_ref[...], wr_ref[...], preferred_element_type=jnp.float32)
         + bl_ref[...])
    out_ref[...] = jnp.maximum(h, 0.0)


def _layer2_tc(sum_ref, inv_ref, h_ref, wl_ref, bl_ref, wr_ref, wo_ref,
               bo_ref, out_ref):
    mean = (sum_ref[0] + sum_ref[1]) * inv_ref[...]
    h = (jnp.dot(mean, wl_ref[...], preferred_element_type=jnp.float32)
         + jnp.dot(h_ref[...], wr_ref[...], preferred_element_type=jnp.float32)
         + bl_ref[...])
    h = jnp.maximum(h, 0.0)
    out_ref[...] = (jnp.dot(h, wo_ref[...], preferred_element_type=jnp.float32)
                    + bo_ref[...])


def _row_block(d):
    return pl.BlockSpec((BN, d), lambda i: (i, 0))


def _part_block(d):
    return pl.BlockSpec((NC, BN, d), lambda i: (0, i, 0))


def _full_block(a, b):
    return pl.BlockSpec((a, b), lambda i: (0, 0))


_GRID = (pl.cdiv(N_NODES, BN),)

_layer1_call = pl.pallas_call(
    _layer1_tc,
    grid=_GRID,
    in_specs=[_part_block(D), _row_block(1), _row_block(D_IN),
              _full_block(D_IN, D_HID), _full_block(1, D_HID),
              _full_block(D_IN, D_HID)],
    out_specs=_row_block(D_HID),
    out_shape=jax.ShapeDtypeStruct((N_NODES, D_HID), jnp.float32),
)

_layer2_call = pl.pallas_call(
    _layer2_tc,
    grid=_GRID,
    in_specs=[_part_block(D), _row_block(1), _row_block(D_HID),
              _full_block(D_HID, D_HID), _full_block(1, D_HID),
              _full_block(D_HID, D_HID), _full_block(D_HID, D_OUT),
              _full_block(1, D_OUT)],
    out_specs=_row_block(D_OUT),
    out_shape=jax.ShapeDtypeStruct((N_NODES, D_OUT), jnp.float32),
)


def kernel(x, edge_index, W_l1, b_l1, W_r1, W_l2, b_l2, W_r2, W_out, b_out):
    ei = edge_index.astype(jnp.int32)
    npad = E_PAD - N_EDGES
    src_p = jnp.concatenate([ei[0], jnp.zeros((npad,), jnp.int32)])
    dst_p = jnp.concatenate([ei[1], jnp.full((npad,), TRASH, jnp.int32)])

    sum1, cnt = _agg_with_counts(x, src_p, dst_p)
    inv = (1.0 / jnp.clip(cnt[0, :, 0] + cnt[1, :, 0], 1.0, None))
    inv = inv.reshape(NA, 1)
    h1 = _layer1_call(sum1, inv, x, W_l1, b_l1.reshape(1, D_HID), W_r1)

    (sum2,) = _agg_no_counts(h1, src_p, dst_p)
    out = _layer2_call(sum2, inv, h1, W_l2, b_l2.reshape(1, D_HID), W_r2,
                       W_out, b_out.reshape(1, D_OUT))
    return out


# speedup vs baseline: 1.0905x; 1.0905x over previous
import jax
import jax.numpy as jnp
from jax import lax
from jax.experimental import pallas as pl
from jax.experimental.pallas import tpu as pltpu
from jax.experimental.pallas import tpu_sc as plsc

N_NODES = 10000
N_EDGES = 320000
D_IN = 128
D_HID = 128
D_OUT = 112
D = 128

NC = 2
NS = 16
NW = NC * NS
NA = 10240
TRASH = 10200
RCH = 64
RCHUNKS = 10
CHUNK = 128
NCHUNKS = 79
EPW = CHUNK * NCHUNKS
E_PAD = EPW * NW


def _fill(ref, nrows, ncols, val):
    v = jnp.full((16,), val, jnp.float32)

    def body(r, _):
        for j in range(ncols // 16):
            ref[r, pl.ds(j * 16, 16)] = v
        return 0

    lax.fori_loop(0, nrows, body, 0)


def _make_agg(with_counts):
    out_type = [jax.ShapeDtypeStruct((NC, NA, D), jnp.float32)]
    if with_counts:
        out_type.append(jax.ShapeDtypeStruct((NC, NA, D), jnp.float32))

    scratch = [
        pltpu.VMEM((CHUNK,), jnp.int32),
        pltpu.VMEM((CHUNK,), jnp.int32),
        pltpu.VMEM((CHUNK, D), jnp.float32),
        pltpu.VMEM_SHARED((NA, D), jnp.float32),
        pltpu.SemaphoreType.DMA,
    ]

    def body(table, src, dst, *rest):
        if with_counts:
            (sum_out, cnt_out, src_v, dst_v, rows_v, acc, gsem) = rest
        else:
            (sum_out, src_v, dst_v, rows_v, acc, gsem) = rest

        sid = lax.axis_index("s")
        cid = lax.axis_index("c")
        wid = cid * NS + sid
        ebase = wid * EPW

        def zero_acc():
            for k in range(RCHUNKS):
                roff = (sid * RCHUNKS + k) * RCH
                pltpu.sync_copy(rows_v.at[pl.ds(0, RCH)],
                                acc.at[pl.ds(roff, RCH)])

        def write_acc(out):
            for k in range(RCHUNKS):
                roff = (sid * RCHUNKS + k) * RCH
                pltpu.sync_copy(acc.at[pl.ds(roff, RCH)],
                                rows_v.at[pl.ds(0, RCH)])
                pltpu.sync_copy(rows_v.at[pl.ds(0, RCH)],
                                out.at[cid, pl.ds(roff, RCH)])

        _fill(rows_v, CHUNK, D, 0.0)
        zero_acc()
        plsc.subcore_barrier()

        if with_counts:
            _fill(rows_v, CHUNK, D, 1.0)

            def cnt_body(c, _):
                off = ebase + c * CHUNK
                pltpu.sync_copy(dst.at[pl.ds(off, CHUNK)], dst_v)
                pltpu.sync_copy(rows_v, acc.at[dst_v], add=True)
                return 0

            lax.fori_loop(0, NCHUNKS, cnt_body, 0)
            plsc.subcore_barrier()
            write_acc(cnt_out)
            _fill(rows_v, CHUNK, D, 0.0)
            zero_acc()
            plsc.subcore_barrier()

        def chunk_body(c, _):
            off = ebase + c * CHUNK
            pltpu.sync_copy(src.at[pl.ds(off, CHUNK)], src_v)
            pltpu.sync_copy(dst.at[pl.ds(off, CHUNK)], dst_v)
            pltpu.async_copy(table.at[src_v], rows_v, gsem).wait()
            pltpu.sync_copy(rows_v, acc.at[dst_v], add=True)
            return 0

        lax.fori_loop(0, NCHUNKS, chunk_body, 0)
        plsc.subcore_barrier()
        write_acc(sum_out)

    mesh = plsc.VectorSubcoreMesh(core_axis_name="c", subcore_axis_name="s")
    return pl.kernel(body, out_type=out_type, mesh=mesh,
                     scratch_types=scratch)


_agg_with_counts = _make_agg(True)
_agg_no_counts = _make_agg(False)


BN = 256


def _layer1_tc(sum_ref, inv_ref, x_ref, wl_ref, bl_ref, wr_ref, out_ref):
    mean = (sum_ref[0] + sum_ref[1]) * inv_ref[...]
    h = (jnp.dot(mean, wl_ref[...], preferred_element_type=jnp.float32)
         + jnp.dot(x_ref[...], wr_ref[...], preferred_element_type=jnp.float32)
         + bl_ref[...])
    out_ref[...] = jnp.maximum(h, 0.0)


def _layer2_tc(sum_ref, inv_ref, h_ref, wl_ref, bl_ref, wr_ref, wo_ref,
               bo_ref, out_ref):
    mean = (sum_ref[0] + sum_ref[1]) * inv_ref[...]
    h = (jnp.dot(mean, wl_ref[...], preferred_element_type=jnp.float32)
         + jnp.dot(h_ref[...], wr_ref[...], preferred_element_type=jnp.float32)
         + bl_ref[...])
    h = jnp.maximum(h, 0.0)
    out_ref[...] = (jnp.dot(h, wo_ref[...], preferred_element_type=jnp.float32)
                    + bo_ref[...])


def _row_block(d):
    return pl.BlockSpec((BN, d), lambda i: (i, 0))


def _part_block(d):
    return pl.BlockSpec((NC, BN, d), lambda i: (0, i, 0))


def _full_block(a, b):
    return pl.BlockSpec((a, b), lambda i: (0, 0))


_GRID = (pl.cdiv(N_NODES, BN),)

_layer1_call = pl.pallas_call(
    _layer1_tc,
    grid=_GRID,
    in_specs=[_part_block(D), _row_block(1), _row_block(D_IN),
              _full_block(D_IN, D_HID), _full_block(1, D_HID),
              _full_block(D_IN, D_HID)],
    out_specs=_row_block(D_HID),
    out_shape=jax.ShapeDtypeStruct((N_NODES, D_HID), jnp.float32),
)

_layer2_call = pl.pallas_call(
    _layer2_tc,
    grid=_GRID,
    in_specs=[_part_block(D), _row_block(1), _row_block(D_HID),
              _full_block(D_HID, D_HID), _full_block(1, D_HID),
              _full_block(D_HID, D_HID), _full_block(D_HID, D_OUT),
              _full_block(1, D_OUT)],
    out_specs=_row_block(D_OUT),
    out_shape=jax.ShapeDtypeStruct((N_NODES, D_OUT), jnp.float32),
)


def kernel(x, edge_index, W_l1, b_l1, W_r1, W_l2, b_l2, W_r2, W_out, b_out):
    ei = edge_index.astype(jnp.int32)
    npad = E_PAD - N_EDGES
    src_p = jnp.concatenate([ei[0], jnp.zeros((npad,), jnp.int32)])
    dst_p = jnp.concatenate([ei[1], jnp.full((npad,), TRASH, jnp.int32)])

    sum1, cnt = _agg_with_counts(x, src_p, dst_p)
    inv = (1.0 / jnp.clip(cnt[0, :, 0] + cnt[1, :, 0], 1.0, None))
    inv = inv.reshape(NA, 1)
    h1 = _layer1_call(sum1, inv, x, W_l1, b_l1.reshape(1, D_HID), W_r1)

    (sum2,) = _agg_no_counts(h1, src_p, dst_p)
    out = _layer2_call(sum2, inv, h1, W_l2, b_l2.reshape(1, D_HID), W_r2,
                       W_out, b_out.reshape(1, D_OUT))
    return out


# speedup vs baseline: 1.1093x; 1.0173x over previous
import jax
import jax.numpy as jnp
from jax import lax
from jax.experimental import pallas as pl
from jax.experimental.pallas import tpu as pltpu
from jax.experimental.pallas import tpu_sc as plsc

N_NODES = 10000
N_EDGES = 320000
D_IN = 128
D_HID = 128
D_OUT = 112
D = 128

NC = 2
NS = 16
NW = NC * NS
NA = 10240
TRASH = 10200
RCH = 64
RCHUNKS = 10
CHUNK = 96
NCHUNKS = 105
EPW = CHUNK * NCHUNKS
E_PAD = EPW * NW


def _fill(ref, nrows, ncols, val):
    v = jnp.full((16,), val, jnp.float32)

    def body(r, _):
        for j in range(ncols // 16):
            ref[r, pl.ds(j * 16, 16)] = v
        return 0

    lax.fori_loop(0, nrows, body, 0)


def _make_agg(with_counts):
    out_type = [jax.ShapeDtypeStruct((NC, NA, D), jnp.float32)]
    if with_counts:
        out_type.append(jax.ShapeDtypeStruct((NC, NA, D), jnp.float32))

    scratch = [
        pltpu.VMEM((CHUNK,), jnp.int32),
        pltpu.VMEM((CHUNK,), jnp.int32),
        pltpu.VMEM((CHUNK, D), jnp.float32),
        pltpu.VMEM_SHARED((NA, D), jnp.float32),
        pltpu.SemaphoreType.DMA,
    ]

    def body(table, src, dst, *rest):
        if with_counts:
            (sum_out, cnt_out, src_v, dst_v, rows_v, acc, gsem) = rest
        else:
            (sum_out, src_v, dst_v, rows_v, acc, gsem) = rest

        sid = lax.axis_index("s")
        cid = lax.axis_index("c")
        wid = cid * NS + sid
        ebase = wid * EPW

        def zero_acc():
            for k in range(RCHUNKS):
                roff = (sid * RCHUNKS + k) * RCH
                pltpu.sync_copy(rows_v.at[pl.ds(0, RCH)],
                                acc.at[pl.ds(roff, RCH)])

        def write_acc(out):
            for k in range(RCHUNKS):
                roff = (sid * RCHUNKS + k) * RCH
                pltpu.sync_copy(acc.at[pl.ds(roff, RCH)],
                                rows_v.at[pl.ds(0, RCH)])
                pltpu.sync_copy(rows_v.at[pl.ds(0, RCH)],
                                out.at[cid, pl.ds(roff, RCH)])

        _fill(rows_v, CHUNK, D, 0.0)
        zero_acc()
        plsc.subcore_barrier()

        if with_counts:
            _fill(rows_v, CHUNK, D, 1.0)

            def cnt_body(c, _):
                off = ebase + c * CHUNK
                pltpu.sync_copy(dst.at[pl.ds(off, CHUNK)], dst_v)
                pltpu.sync_copy(rows_v, acc.at[dst_v], add=True)
                return 0

            lax.fori_loop(0, NCHUNKS, cnt_body, 0)
            plsc.subcore_barrier()
            write_acc(cnt_out)
            _fill(rows_v, CHUNK, D, 0.0)
            zero_acc()
            plsc.subcore_barrier()

        def chunk_body(c, _):
            off = ebase + c * CHUNK
            pltpu.sync_copy(src.at[pl.ds(off, CHUNK)], src_v)
            pltpu.sync_copy(dst.at[pl.ds(off, CHUNK)], dst_v)
            pltpu.async_copy(table.at[src_v], rows_v, gsem).wait()
            pltpu.sync_copy(rows_v, acc.at[dst_v], add=True)
            return 0

        lax.fori_loop(0, NCHUNKS, chunk_body, 0)
        plsc.subcore_barrier()
        write_acc(sum_out)

    mesh = plsc.VectorSubcoreMesh(core_axis_name="c", subcore_axis_name="s")
    return pl.kernel(body, out_type=out_type, mesh=mesh,
                     scratch_types=scratch)


_agg_with_counts = _make_agg(True)
_agg_no_counts = _make_agg(False)


BN = 256


def _layer1_tc(sum_ref, inv_ref, x_ref, wl_ref, bl_ref, wr_ref, out_ref):
    mean = (sum_ref[0] + sum_ref[1]) * inv_ref[...]
    h = (jnp.dot(mean, wl_ref[...], preferred_element_type=jnp.float32)
         + jnp.dot(x_ref[...], wr_ref[...], preferred_element_type=jnp.float32)
         + bl_ref[...])
    out_ref[...] = jnp.maximum(h, 0.0)


def _layer2_tc(sum_ref, inv_ref, h_ref, wl_ref, bl_ref, wr_ref, wo_ref,
               bo_ref, out_ref):
    mean = (sum_ref[0] + sum_ref[1]) * inv_ref[...]
    h = (jnp.dot(mean, wl_ref[...], preferred_element_type=jnp.float32)
         + jnp.dot(h_ref[...], wr_ref[...], preferred_element_type=jnp.float32)
         + bl_ref[...])
    h = jnp.maximum(h, 0.0)
    out_ref[...] = (jnp.dot(h, wo_ref[...], preferred_element_type=jnp.float32)
                    + bo_ref[...])


def _row_block(d):
    return pl.BlockSpec((BN, d), lambda i: (i, 0))


def _part_block(d):
    return pl.BlockSpec((NC, BN, d), lambda i: (0, i, 0))


def _full_block(a, b):
    return pl.BlockSpec((a, b), lambda i: (0, 0))


_GRID = (pl.cdiv(N_NODES, BN),)

_layer1_call = pl.pallas_call(
    _layer1_tc,
    grid=_GRID,
    in_specs=[_part_block(D), _row_block(1), _row_block(D_IN),
              _full_block(D_IN, D_HID), _full_block(1, D_HID),
              _full_block(D_IN, D_HID)],
    out_specs=_row_block(D_HID),
    out_shape=jax.ShapeDtypeStruct((N_NODES, D_HID), jnp.float32),
)

_layer2_call = pl.pallas_call(
    _layer2_tc,
    grid=_GRID,
    in_specs=[_part_block(D), _row_block(1), _row_block(D_HID),
              _full_block(D_HID, D_HID), _full_block(1, D_HID),
              _full_block(D_HID, D_HID), _full_block(D_HID, D_OUT),
              _full_block(1, D_OUT)],
    out_specs=_row_block(D_OUT),
    out_shape=jax.ShapeDtypeStruct((N_NODES, D_OUT), jnp.float32),
)


def kernel(x, edge_index, W_l1, b_l1, W_r1, W_l2, b_l2, W_r2, W_out, b_out):
    ei = edge_index.astype(jnp.int32)
    npad = E_PAD - N_EDGES
    src_p = jnp.concatenate([ei[0], jnp.zeros((npad,), jnp.int32)])
    dst_p = jnp.concatenate([ei[1], jnp.full((npad,), TRASH, jnp.int32)])

    sum1, cnt = _agg_with_counts(x, src_p, dst_p)
    inv = (1.0 / jnp.clip(cnt[0, :, 0] + cnt[1, :, 0], 1.0, None))
    inv = inv.reshape(NA, 1)
    h1 = _layer1_call(sum1, inv, x, W_l1, b_l1.reshape(1, D_HID), W_r1)

    (sum2,) = _agg_no_counts(h1, src_p, dst_p)
    out = _layer2_call(sum2, inv, h1, W_l2, b_l2.reshape(1, D_HID), W_r2,
                       W_out, b_out.reshape(1, D_OUT))
    return out
